# baseline (speedup 1.0000x reference)
import functools

import jax
import jax.numpy as jnp
from jax import lax
from jax.experimental import pallas as pl
from jax.experimental.pallas import tpu as pltpu
from jax.experimental.pallas import tpu_sc as plsc

B = 4096
L = 200
D = 64
NW = 32


def _build(batch, seq_len, d_model, num_workers, interpret=False):
    rpw = batch // num_workers

    c0 = min(112, seq_len)
    c1r = seq_len - c0
    c1 = ((c1r + 15) // 16) * 16
    lp = c0 + c1
    nch = lp // 16
    nc0 = c0 // 16

    def body(inputs_hbm, emb_hbm, w_hbm, out_hbm,
             idx0_v, idx1_v, rows_v, w_v, out_v, sem):
        if interpret:
            wid = 0
        else:
            wid = lax.axis_index("s") * 2 + lax.axis_index("c")
        base = pl.multiple_of(wid * rpw, 8)

        if c1 > c1r:
            idx1_v[pl.ds(c1 - 16, 16)] = jnp.zeros((16,), jnp.int32)

        @pl.loop(0, rpw)
        def _row(r):
            b = base + r
            o = pl.multiple_of(b * seq_len, 8)
            pltpu.sync_copy(inputs_hbm.at[pl.ds(o, c0)], idx0_v)
            o1 = pl.multiple_of(b * seq_len + c0, 8)
            pltpu.sync_copy(inputs_hbm.at[pl.ds(o1, c1r)],
                            idx1_v.at[pl.ds(0, c1r)])
            cps = [
                pltpu.async_copy(emb_hbm.at[idx0_v],
                                 rows_v.at[pl.ds(0, c0)], sem),
                pltpu.async_copy(emb_hbm.at[idx1_v],
                                 rows_v.at[pl.ds(c0, c1)], sem),
                pltpu.async_copy(w_hbm.at[idx0_v],
                                 w_v.at[pl.ds(0, c0)], sem),
                pltpu.async_copy(w_hbm.at[idx1_v],
                                 w_v.at[pl.ds(c0, c1)], sem),
            ]
            for cp in cps:
                cp.wait()

            nacc = d_model // 16
            accs = [jnp.zeros((16,), jnp.float32)] * (2 * nacc)
            for k in range(nch):
                if k < nc0:
                    iv = idx0_v[pl.ds(16 * k, 16)]
                else:
                    iv = idx1_v[pl.ds(16 * (k - nc0), 16)]
                wv = w_v[pl.ds(16 * k, 16)]
                wmv = jnp.where(iv != 0, wv, jnp.zeros((16,), jnp.float32))
                for j in range(16):
                    w = wmv[j]
                    l = 16 * k + j
                    p = j % 2
                    for c in range(nacc):
                        accs[2 * c + p] = (accs[2 * c + p]
                                           + rows_v[l, pl.ds(16 * c, 16)] * w)
            for c in range(nacc):
                out_v[r, pl.ds(16 * c, 16)] = accs[2 * c] + accs[2 * c + 1]

        pltpu.sync_copy(out_v, out_hbm.at[pl.ds(base, rpw)])

    return pl.kernel(
        body,
        out_type=jax.ShapeDtypeStruct((batch, d_model), jnp.float32),
        mesh=plsc.VectorSubcoreMesh(core_axis_name="c",
                                    subcore_axis_name="s",
                                    num_cores=2, num_subcores=16),
        scratch_types=[
            pltpu.VMEM((c0,), jnp.int32),
            pltpu.VMEM((c1,), jnp.int32),
            pltpu.VMEM((lp, d_model), jnp.float32),
            pltpu.VMEM((lp,), jnp.float32),
            pltpu.VMEM((rpw, d_model), jnp.float32),
            pltpu.SemaphoreType.DMA,
        ],
        compiler_params=pltpu.CompilerParams(use_tc_tiling_on_sc=False),
        interpret=interpret,
    )


_sc_kernel = _build(B, L, D, NW)


def kernel(inputs, emb_table, weight_table):
    return _sc_kernel(inputs.reshape(-1), emb_table,
                      weight_table.reshape(-1))
